# baseline (speedup 1.0000x reference)
import functools

import jax
import jax.numpy as jnp
import numpy as np
from jax import lax
from jax.experimental import pallas as pl
from jax.experimental.pallas import tpu as pltpu
from jax.experimental.pallas import tpu_sc as plsc

N_LEVELS = 16
F_PER_LEVEL = 2
TABLE_SIZE = 2 ** 19
_GROWTH = np.exp((np.log(4096.0) - np.log(16.0)) / (N_LEVELS - 1))
_SCALINGS = np.floor(16.0 * _GROWTH ** np.arange(N_LEVELS)).astype(np.float32)

_P1 = np.int32(2654435761 - 2 ** 32)
_P2 = np.int32(805459861)
_MASK = np.int32(TABLE_SIZE - 1)

NC = 2
NS = 16
NW = NC * NS
LANES = 16

B = 131072
NF = N_LEVELS * F_PER_LEVEL
CHUNK = 128
GROUPS = CHUNK // LANES
PER_W = B // NW
NCHUNK = PER_W // CHUNK
NIDX = CHUNK * N_LEVELS * 8


def _make_kernel():
    mesh = plsc.VectorSubcoreMesh(core_axis_name="c", subcore_axis_name="s")

    @functools.partial(
        pl.kernel,
        mesh=mesh,
        compiler_params=pltpu.CompilerParams(needs_layout_passes=False),
        out_type=jax.ShapeDtypeStruct((B * NF,), jnp.float32),
        scratch_types=[
            pltpu.VMEM((CHUNK,), jnp.float32),
            pltpu.VMEM((CHUNK,), jnp.float32),
            pltpu.VMEM((CHUNK,), jnp.float32),
            pltpu.VMEM((N_LEVELS * CHUNK,), jnp.float32),
            pltpu.VMEM((N_LEVELS * CHUNK,), jnp.float32),
            pltpu.VMEM((N_LEVELS * CHUNK,), jnp.float32),
            pltpu.VMEM((NIDX,), jnp.int32),
            pltpu.VMEM((NIDX,), jnp.int32),
            pltpu.VMEM((NIDX,), jnp.float32),
            pltpu.VMEM((NIDX,), jnp.float32),
            pltpu.VMEM((CHUNK * NF,), jnp.float32),
            pltpu.SemaphoreType.DMA,
            pltpu.SemaphoreType.DMA,
        ],
    )
    def sc_kernel(xs_hbm, ys_hbm, zs_hbm, table_hbm, out_hbm,
                  xv, yv, zv, oxv, oyv, ozv, idx0v, idx1v,
                  rows0v, rows1v, outv, sem0, sem1):
        wid = lax.axis_index("s") * NC + lax.axis_index("c")
        iota = lax.iota(jnp.int32, LANES)

        def chunk_body(c, carry):
            pbase = wid * PER_W + c * CHUNK
            pltpu.sync_copy(xs_hbm.at[pl.ds(pbase, CHUNK)], xv)
            pltpu.sync_copy(ys_hbm.at[pl.ds(pbase, CHUNK)], yv)
            pltpu.sync_copy(zs_hbm.at[pl.ds(pbase, CHUNK)], zv)

            def hash_group(g, carry):
                s = g * LANES
                x16 = xv[pl.ds(s, LANES)]
                y16 = yv[pl.ds(s, LANES)]
                z16 = zv[pl.ds(s, LANES)]
                for l in range(N_LEVELS):
                    sl = float(_SCALINGS[l])
                    sx = x16 * sl
                    sy = y16 * sl
                    sz = z16 * sl
                    xf = sx.astype(jnp.int32)
                    yf = sy.astype(jnp.int32)
                    zf = sz.astype(jnp.int32)
                    xff = xf.astype(jnp.float32)
                    yff = yf.astype(jnp.float32)
                    zff = zf.astype(jnp.float32)
                    oxv[pl.ds(l * CHUNK + s, LANES)] = sx - xff
                    oyv[pl.ds(l * CHUNK + s, LANES)] = sy - yff
                    ozv[pl.ds(l * CHUNK + s, LANES)] = sz - zff
                    xc = jnp.where(sx > xff, xf + 1, xf)
                    yc = jnp.where(sy > yff, yf + 1, yf)
                    zc = jnp.where(sz > zff, zf + 1, zf)
                    hyf = yf * _P1
                    hyc = yc * _P1
                    hzf = zf * _P2
                    hzc = zc * _P2
                    ycc = hyc ^ hzc
                    yfc = hyf ^ hzc
                    yff_ = hyf ^ hzf
                    ycf = hyc ^ hzf
                    lofs = np.int32(l * TABLE_SIZE)
                    rb = g * (N_LEVELS * 128) + l * 128
                    combos = (xc ^ ycc, xc ^ yfc, xf ^ yfc, xf ^ ycc,
                              xc ^ ycf, xc ^ yff_, xf ^ yff_, xf ^ ycf)
                    for corner, hv in enumerate(combos):
                        h = ((hv & _MASK) + lofs)
                        e = h + h
                        idx0v[pl.ds(rb + corner * LANES, LANES)] = e
                        idx1v[pl.ds(rb + corner * LANES, LANES)] = e + 1
                return carry

            lax.fori_loop(np.int32(0), np.int32(GROUPS), hash_group,
                          np.int32(0))

            cp0 = pltpu.async_copy(table_hbm.at[idx0v], rows0v, sem0)
            cp1 = pltpu.async_copy(table_hbm.at[idx1v], rows1v, sem1)
            cp0.wait()
            cp1.wait()

            def interp_group(g, carry):
                s = g * LANES
                p32 = (iota + s) * NF
                for l in range(N_LEVELS):
                    ox = oxv[pl.ds(l * CHUNK + s, LANES)]
                    oy = oyv[pl.ds(l * CHUNK + s, LANES)]
                    oz = ozv[pl.ds(l * CHUNK + s, LANES)]
                    mx = 1.0 - ox
                    my = 1.0 - oy
                    mz = 1.0 - oz
                    rb = g * (N_LEVELS * 128) + l * 128
                    for ft, rv in ((0, rows0v), (1, rows1v)):
                        f = [rv[pl.ds(rb + corner * LANES, LANES)]
                             for corner in range(8)]
                        f03 = f[0] * ox + f[3] * mx
                        f12 = f[1] * ox + f[2] * mx
                        f56 = f[5] * ox + f[6] * mx
                        f47 = f[4] * ox + f[7] * mx
                        f0312 = f03 * oy + f12 * my
                        f4756 = f47 * oy + f56 * my
                        enc = f0312 * oz + f4756 * mz
                        plsc.store_scatter(outv, [p32 + (2 * l + ft)], enc)
                return carry

            lax.fori_loop(np.int32(0), np.int32(GROUPS), interp_group,
                          np.int32(0))

            pltpu.sync_copy(outv, out_hbm.at[pl.ds(pbase * NF, CHUNK * NF)])
            return carry

        lax.fori_loop(np.int32(0), np.int32(NCHUNK), chunk_body, np.int32(0))

    return sc_kernel


_sc_kernel_cache = []


@jax.jit
def _run(in_tensor, hash_table):
    if not _sc_kernel_cache:
        _sc_kernel_cache.append(_make_kernel())
    coords = in_tensor.T
    flat = _sc_kernel_cache[0](coords[0], coords[1], coords[2],
                               hash_table.reshape(-1))
    return flat.reshape(B, NF)


def kernel(in_tensor, hash_table):
    with jax.enable_x64(False):
        return _run(in_tensor, hash_table)
